# baseline (speedup 1.0000x reference)
import functools

import jax
import jax.numpy as jnp
from jax import lax
from jax.experimental import pallas as pl
from jax.experimental.pallas import tpu as pltpu
from jax.experimental.pallas import tpu_sc as plsc

X = 512
Y = 512
NCELL = X * Y
B, V, C = 2, 20000, 64
VPAD = 20008
L = 16
NC, NS = 2, 16
SLAB = NCELL // NS
CCH = 2000
NCCH = V // CCH
CHUNK = 8192
NCHUNK = NCELL // CHUNK

_mesh = plsc.VectorSubcoreMesh(
    core_axis_name="c", subcore_axis_name="s", num_cores=NC, num_subcores=NS
)
_sc_params = pltpu.CompilerParams(
    needs_layout_passes=False, use_tc_tiling_on_sc=False
)


def _transpose_body(f_ref, o_ref):
    o_ref[...] = f_ref[...].T


def _transpose(feat_pad):
    return pl.pallas_call(
        _transpose_body,
        grid=(B,),
        in_specs=[pl.BlockSpec((None, VPAD, C), lambda b: (b, 0, 0))],
        out_specs=pl.BlockSpec((None, C, VPAD), lambda b: (b, 0, 0)),
        out_shape=jax.ShapeDtypeStruct((B, C, VPAD), jnp.float32),
    )(feat_pad)


@functools.partial(
    pl.kernel,
    out_type=(
        jax.ShapeDtypeStruct((B, C, NCELL), jnp.float32),
        jax.ShapeDtypeStruct((NC, NCELL), jnp.int16),
    ),
    mesh=_mesh,
    compiler_params=_sc_params,
    scratch_types=[
        pltpu.VMEM((2, CCH * 3), jnp.int32),
        pltpu.VMEM((SLAB,), jnp.int32),
        pltpu.VMEM((SLAB,), jnp.int16),
        pltpu.VMEM((VPAD,), jnp.float32),
        pltpu.VMEM((VPAD,), jnp.float32),
        pltpu.VMEM((2, CHUNK), jnp.int16),
        pltpu.VMEM((2, 2, CHUNK), jnp.float32),
        pltpu.SemaphoreType.DMA,
        pltpu.SemaphoreType.DMA,
        pltpu.SemaphoreType.DMA,
        pltpu.SemaphoreType.DMA,
    ],
)
def _scatter_kernel(
    featT_hbm, coords_hbm, out_hbm, wgrid_hbm,
    cbuf, wslab, wpack, ft0, ft1, wbuf, obuf,
    sem_c, sem_ft, sem_w, sem_o,
):
    cid = lax.axis_index("c")
    sid = lax.axis_index("s")
    ch0 = 2 * (cid * NS + sid)
    base = sid * SLAB
    lanes = lax.iota(jnp.int32, L)

    pltpu.async_copy(featT_hbm.at[0, ch0], ft0, sem_ft)
    pltpu.async_copy(featT_hbm.at[0, ch0 + 1], ft1, sem_ft)

    for b in range(B):
        def initf(j, carry):
            for u in range(8):
                wslab[pl.ds((j * 8 + u) * L, L)] = jnp.full((L,), -1, jnp.int32)
            return carry

        lax.fori_loop(0, SLAB // (8 * L), initf, 0)

        pltpu.async_copy(coords_hbm.at[b, pl.ds(0, CCH * 3)], cbuf.at[0], sem_c)
        for k in range(NCCH):
            slot = k % 2
            pltpu.make_async_copy(
                coords_hbm.at[b, pl.ds(0, CCH * 3)], cbuf.at[slot], sem_c
            ).wait()
            if k + 1 < NCCH:
                pltpu.async_copy(
                    coords_hbm.at[b, pl.ds((k + 1) * CCH * 3, CCH * 3)],
                    cbuf.at[(k + 1) % 2],
                    sem_c,
                )

            def grp(i, carry):
                vidx3 = (i * L + lanes) * 3
                xs = plsc.load_gather(cbuf.at[slot], [vidx3])
                ys = plsc.load_gather(cbuf.at[slot], [vidx3 + 1])
                li = ys * X + xs - base
                m = (li >= 0) & (li < SLAB)
                li_safe = jnp.clip(li, 0, SLAB - 1)
                vidx = (k * CCH + i * L) + lanes

                @pl.when(jnp.any(m))
                def _():
                    def body(_):
                        cur = plsc.load_gather(wslab, [li_safe])
                        need = m & (cur < vidx)
                        plsc.store_scatter(
                            wslab, [li_safe], jnp.maximum(cur, vidx), mask=need
                        )
                        return jnp.any(need)

                    lax.while_loop(lambda c2: c2, body, jnp.bool_(True))

                return carry

            lax.fori_loop(0, CCH // L, grp, 0)

        def flushf(j, carry):
            for u in range(4):
                o = (j * 4 + u) * 2 * L
                w0 = wslab[pl.ds(o, L)]
                w1 = wslab[pl.ds(o + L, L)]
                w0 = jnp.where(w0 < 0, V, w0)
                w1 = jnp.where(w1 < 0, V, w1)
                wpack[pl.ds(o, 2 * L)] = plsc.pack(
                    w0, w1, format=plsc.PackFormat.INTERLEAVED
                )
            return carry

        lax.fori_loop(0, SLAB // (8 * L), flushf, 0)
        pltpu.sync_copy(wpack, wgrid_hbm.at[cid, pl.ds(base, SLAB)])

        plsc.subcore_barrier()

        pltpu.make_async_copy(featT_hbm.at[b, ch0], ft0, sem_ft).wait()
        pltpu.make_async_copy(featT_hbm.at[b, ch0 + 1], ft1, sem_ft).wait()

        pltpu.async_copy(wgrid_hbm.at[cid, pl.ds(0, CHUNK)], wbuf.at[0], sem_w)
        for k in range(NCHUNK):
            slot = k % 2
            off = k * CHUNK
            t = b * NCHUNK + k
            pltpu.make_async_copy(
                wgrid_hbm.at[cid, pl.ds(0, CHUNK)], wbuf.at[slot], sem_w
            ).wait()
            if k + 1 < NCHUNK:
                pltpu.async_copy(
                    wgrid_hbm.at[cid, pl.ds((k + 1) * CHUNK, CHUNK)],
                    wbuf.at[(k + 1) % 2],
                    sem_w,
                )
            if t >= 2:
                pltpu.make_async_copy(
                    obuf.at[slot, 0], out_hbm.at[b, ch0, pl.ds(off, CHUNK)],
                    sem_o,
                ).wait()
                pltpu.make_async_copy(
                    obuf.at[slot, 1], out_hbm.at[b, ch0, pl.ds(off, CHUNK)],
                    sem_o,
                ).wait()

            def grp(j, carry):
                for u in range(4):
                    o = (j * 4 + u) * 2 * L
                    w16 = wbuf[slot, pl.ds(o, 2 * L)]
                    g0, g1 = plsc.unpack(
                        w16, format=plsc.PackFormat.INTERLEAVED
                    )
                    obuf[slot, 0, pl.ds(o, L)] = plsc.load_gather(ft0, [g0])
                    obuf[slot, 0, pl.ds(o + L, L)] = plsc.load_gather(ft0, [g1])
                    obuf[slot, 1, pl.ds(o, L)] = plsc.load_gather(ft1, [g0])
                    obuf[slot, 1, pl.ds(o + L, L)] = plsc.load_gather(ft1, [g1])
                return carry

            lax.fori_loop(0, CHUNK // (8 * L), grp, 0)
            pltpu.async_copy(
                obuf.at[slot, 0], out_hbm.at[b, ch0, pl.ds(off, CHUNK)], sem_o
            )
            pltpu.async_copy(
                obuf.at[slot, 1], out_hbm.at[b, ch0 + 1, pl.ds(off, CHUNK)],
                sem_o,
            )

        if b == 0:
            pltpu.async_copy(featT_hbm.at[1, ch0], ft0, sem_ft)
            pltpu.async_copy(featT_hbm.at[1, ch0 + 1], ft1, sem_ft)
            plsc.subcore_barrier()

    for _ in range(4):
        pltpu.make_async_copy(
            obuf.at[0, 0], out_hbm.at[B - 1, ch0, pl.ds(0, CHUNK)], sem_o
        ).wait()


def kernel(pillar_features, coords):
    feat_pad = jnp.pad(pillar_features, ((0, 0), (0, VPAD - V), (0, 0)))
    featT = _transpose(feat_pad)
    out, _ = _scatter_kernel(featT, coords.reshape(B, V * 3))
    return out.reshape(B, C, Y, X)


# speedup vs baseline: 1.4993x; 1.4993x over previous
import functools

import jax
import jax.numpy as jnp
from jax import lax
from jax.experimental import pallas as pl
from jax.experimental.pallas import tpu as pltpu
from jax.experimental.pallas import tpu_sc as plsc

X = 512
Y = 512
NCELL = X * Y
B, V, C = 2, 20000, 64
VPAD = 20008
L = 16
NC, NS = 2, 16
SLAB = NCELL // NS
CCH = 2000
NCCH = V // CCH
CHUNK = 8192
NCHUNK = NCELL // CHUNK

_mesh = plsc.VectorSubcoreMesh(
    core_axis_name="c", subcore_axis_name="s", num_cores=NC, num_subcores=NS
)
_sc_params = pltpu.CompilerParams(
    needs_layout_passes=False, use_tc_tiling_on_sc=False
)


def _transpose_body(f_ref, o_ref):
    o_ref[...] = f_ref[...].T


def _transpose(feat_pad):
    return pl.pallas_call(
        _transpose_body,
        grid=(B,),
        in_specs=[pl.BlockSpec((None, VPAD, C), lambda b: (b, 0, 0))],
        out_specs=pl.BlockSpec((None, C, VPAD), lambda b: (b, 0, 0)),
        out_shape=jax.ShapeDtypeStruct((B, C, VPAD), jnp.float32),
    )(feat_pad)


@functools.partial(
    pl.kernel,
    out_type=(
        jax.ShapeDtypeStruct((B, C, NCELL), jnp.float32),
        jax.ShapeDtypeStruct((NC, NCELL), jnp.int16),
    ),
    mesh=_mesh,
    compiler_params=_sc_params,
    scratch_types=[
        pltpu.VMEM((2, CCH * 3), jnp.int32),
        pltpu.VMEM((SLAB,), jnp.int32),
        pltpu.VMEM((SLAB,), jnp.int16),
        pltpu.VMEM((VPAD,), jnp.float32),
        pltpu.VMEM((VPAD,), jnp.float32),
        pltpu.VMEM((2, CHUNK), jnp.int16),
        pltpu.VMEM((2, 2, CHUNK), jnp.float32),
        pltpu.SemaphoreType.DMA,
        pltpu.SemaphoreType.DMA,
        pltpu.SemaphoreType.DMA,
        pltpu.SemaphoreType.DMA,
    ],
)
def _scatter_kernel(
    featT_hbm, coords_hbm, out_hbm, wgrid_hbm,
    cbuf, wslab, wpack, ft0, ft1, wbuf, obuf,
    sem_c, sem_ft, sem_w, sem_o,
):
    cid = lax.axis_index("c")
    sid = lax.axis_index("s")
    ch0 = 2 * (cid * NS + sid)
    base = sid * SLAB
    lanes = lax.iota(jnp.int32, L)

    pltpu.async_copy(featT_hbm.at[0, ch0], ft0, sem_ft)
    pltpu.async_copy(featT_hbm.at[0, ch0 + 1], ft1, sem_ft)

    for b in range(B):
        @plsc.parallel_loop(0, SLAB, 8 * L, unroll=2)
        def _(o0):
            for u in range(8):
                wslab[pl.ds(o0 + u * L, L)] = jnp.full((L,), -1, jnp.int32)

        pltpu.async_copy(coords_hbm.at[b, pl.ds(0, CCH * 3)], cbuf.at[0], sem_c)

        def coord_pair(q, carry):
            for s in range(2):
                k = q * 2 + s
                pltpu.make_async_copy(
                    coords_hbm.at[b, pl.ds(0, CCH * 3)], cbuf.at[s], sem_c
                ).wait()

                @pl.when(k < NCCH - 1)
                def _():
                    pltpu.async_copy(
                        coords_hbm.at[b, pl.ds((k + 1) * (CCH * 3), CCH * 3)],
                        cbuf.at[1 - s],
                        sem_c,
                    )

                def grp(i, c2):
                    vidx3 = (i * L + lanes) * 3
                    xs = plsc.load_gather(cbuf.at[s], [vidx3])
                    ys = plsc.load_gather(cbuf.at[s], [vidx3 + 1])
                    li = ys * X + xs - base
                    m = (li >= 0) & (li < SLAB)
                    li_safe = jnp.clip(li, 0, SLAB - 1)
                    vidx = (k * CCH + i * L) + lanes

                    @pl.when(jnp.any(m))
                    def _():
                        def body(_):
                            cur = plsc.load_gather(wslab, [li_safe])
                            need = m & (cur < vidx)
                            plsc.store_scatter(
                                wslab, [li_safe], jnp.maximum(cur, vidx),
                                mask=need,
                            )
                            return jnp.any(need)

                        lax.while_loop(lambda c3: c3, body, jnp.bool_(True))

                    return c2

                lax.fori_loop(0, CCH // L, grp, 0)
            return carry

        lax.fori_loop(0, NCCH // 2, coord_pair, 0)

        @plsc.parallel_loop(0, SLAB, 8 * L, unroll=2)
        def _(o0):
            for u in range(4):
                o = o0 + u * 2 * L
                w0 = wslab[pl.ds(o, L)]
                w1 = wslab[pl.ds(o + L, L)]
                w0 = jnp.where(w0 < 0, V, w0)
                w1 = jnp.where(w1 < 0, V, w1)
                wpack[pl.ds(o, 2 * L)] = plsc.pack(
                    w0, w1, format=plsc.PackFormat.INTERLEAVED
                )

        pltpu.sync_copy(wpack, wgrid_hbm.at[cid, pl.ds(base, SLAB)])

        plsc.subcore_barrier()

        pltpu.make_async_copy(featT_hbm.at[b, ch0], ft0, sem_ft).wait()
        pltpu.make_async_copy(featT_hbm.at[b, ch0 + 1], ft1, sem_ft).wait()

        pltpu.async_copy(wgrid_hbm.at[cid, pl.ds(0, CHUNK)], wbuf.at[0], sem_w)

        def emit_pair(q, carry):
            for s in range(2):
                k = q * 2 + s
                off = k * CHUNK
                t = b * NCHUNK + k
                pltpu.make_async_copy(
                    wgrid_hbm.at[cid, pl.ds(0, CHUNK)], wbuf.at[s], sem_w
                ).wait()

                @pl.when(k < NCHUNK - 1)
                def _():
                    pltpu.async_copy(
                        wgrid_hbm.at[cid, pl.ds(off + CHUNK, CHUNK)],
                        wbuf.at[1 - s],
                        sem_w,
                    )

                @pl.when(t >= 2)
                def _():
                    pltpu.make_async_copy(
                        obuf.at[s, 0], out_hbm.at[b, ch0, pl.ds(off, CHUNK)],
                        sem_o,
                    ).wait()
                    pltpu.make_async_copy(
                        obuf.at[s, 1], out_hbm.at[b, ch0, pl.ds(off, CHUNK)],
                        sem_o,
                    ).wait()

                @plsc.parallel_loop(0, CHUNK, 8 * L, unroll=2)
                def _(o0):
                    for u in range(4):
                        o = o0 + u * 2 * L
                        w16 = wbuf[s, pl.ds(o, 2 * L)]
                        g0, g1 = plsc.unpack(
                            w16, format=plsc.PackFormat.INTERLEAVED
                        )
                        obuf[s, 0, pl.ds(o, L)] = plsc.load_gather(ft0, [g0])
                        obuf[s, 0, pl.ds(o + L, L)] = plsc.load_gather(
                            ft0, [g1]
                        )
                        obuf[s, 1, pl.ds(o, L)] = plsc.load_gather(ft1, [g0])
                        obuf[s, 1, pl.ds(o + L, L)] = plsc.load_gather(
                            ft1, [g1]
                        )

                pltpu.async_copy(
                    obuf.at[s, 0], out_hbm.at[b, ch0, pl.ds(off, CHUNK)],
                    sem_o,
                )
                pltpu.async_copy(
                    obuf.at[s, 1], out_hbm.at[b, ch0 + 1, pl.ds(off, CHUNK)],
                    sem_o,
                )
            return carry

        lax.fori_loop(0, NCHUNK // 2, emit_pair, 0)

        if b == 0:
            pltpu.async_copy(featT_hbm.at[1, ch0], ft0, sem_ft)
            pltpu.async_copy(featT_hbm.at[1, ch0 + 1], ft1, sem_ft)
            plsc.subcore_barrier()

    for _ in range(4):
        pltpu.make_async_copy(
            obuf.at[0, 0], out_hbm.at[B - 1, ch0, pl.ds(0, CHUNK)], sem_o
        ).wait()


def kernel(pillar_features, coords):
    feat_pad = jnp.pad(pillar_features, ((0, 0), (0, VPAD - V), (0, 0)))
    featT = _transpose(feat_pad)
    out, _ = _scatter_kernel(featT, coords.reshape(B, V * 3))
    return out.reshape(B, C, Y, X)


# speedup vs baseline: 1.5869x; 1.0584x over previous
import functools

import jax
import jax.numpy as jnp
from jax import lax
from jax.experimental import pallas as pl
from jax.experimental.pallas import tpu as pltpu
from jax.experimental.pallas import tpu_sc as plsc

X = 512
Y = 512
NCELL = X * Y
B, V, C = 2, 20000, 64
VPAD = 20008
L = 16
NC, NS = 2, 16
SLAB = NCELL // NS
CCH = 2000
NCCH = V // CCH
CHUNK = 8192
NCHUNK = NCELL // CHUNK

_mesh = plsc.VectorSubcoreMesh(
    core_axis_name="c", subcore_axis_name="s", num_cores=NC, num_subcores=NS
)
_sc_params = pltpu.CompilerParams(
    needs_layout_passes=False, use_tc_tiling_on_sc=False
)


def _transpose_body(f_ref, o_ref):
    o_ref[...] = f_ref[...].T


def _transpose(feat_pad):
    return pl.pallas_call(
        _transpose_body,
        grid=(B,),
        in_specs=[pl.BlockSpec((None, VPAD, C), lambda b: (b, 0, 0))],
        out_specs=pl.BlockSpec((None, C, VPAD), lambda b: (b, 0, 0)),
        out_shape=jax.ShapeDtypeStruct((B, C, VPAD), jnp.float32),
    )(feat_pad)


@functools.partial(
    pl.kernel,
    out_type=jax.ShapeDtypeStruct((B, C, NCELL), jnp.float32),
    mesh=_mesh,
    compiler_params=_sc_params,
    scratch_types=[
        pltpu.VMEM((2, CCH * 3), jnp.int32),
        pltpu.VMEM((SLAB,), jnp.int32),
        pltpu.VMEM((SLAB,), jnp.int16),
        pltpu.VMEM((VPAD,), jnp.float32),
        pltpu.VMEM((VPAD,), jnp.float32),
        pltpu.VMEM((2, CHUNK), jnp.int16),
        pltpu.VMEM((2, 2, CHUNK), jnp.float32),
        pltpu.VMEM_SHARED((NCELL,), jnp.int16),
        pltpu.SemaphoreType.DMA,
        pltpu.SemaphoreType.DMA,
        pltpu.SemaphoreType.DMA,
        pltpu.SemaphoreType.DMA,
    ],
)
def _scatter_kernel(
    featT_hbm, coords_hbm, out_hbm,
    cbuf, wslab, wpack, ft0, ft1, wbuf, obuf, wshared,
    sem_c, sem_ft, sem_w, sem_o,
):
    cid = lax.axis_index("c")
    sid = lax.axis_index("s")
    ch0 = 2 * (cid * NS + sid)
    base = sid * SLAB
    lanes = lax.iota(jnp.int32, L)

    pltpu.async_copy(featT_hbm.at[0, ch0], ft0, sem_ft)
    pltpu.async_copy(featT_hbm.at[0, ch0 + 1], ft1, sem_ft)

    for b in range(B):
        @plsc.parallel_loop(0, SLAB, 8 * L, unroll=2)
        def _(o0):
            for u in range(8):
                wslab[pl.ds(o0 + u * L, L)] = jnp.full((L,), -1, jnp.int32)

        pltpu.async_copy(coords_hbm.at[b, pl.ds(0, CCH * 3)], cbuf.at[0], sem_c)

        def coord_pair(q, carry):
            for s in range(2):
                k = q * 2 + s
                pltpu.make_async_copy(
                    coords_hbm.at[b, pl.ds(0, CCH * 3)], cbuf.at[s], sem_c
                ).wait()

                @pl.when(k < NCCH - 1)
                def _():
                    pltpu.async_copy(
                        coords_hbm.at[b, pl.ds((k + 1) * (CCH * 3), CCH * 3)],
                        cbuf.at[1 - s],
                        sem_c,
                    )

                def grp(i, c2):
                    vidx3 = (i * L + lanes) * 3
                    xs = plsc.load_gather(cbuf.at[s], [vidx3])
                    ys = plsc.load_gather(cbuf.at[s], [vidx3 + 1])
                    li = ys * X + xs - base
                    m = (li >= 0) & (li < SLAB)
                    li_safe = jnp.clip(li, 0, SLAB - 1)
                    vidx = (k * CCH + i * L) + lanes

                    @pl.when(jnp.any(m))
                    def _():
                        def body(_):
                            cur = plsc.load_gather(wslab, [li_safe])
                            need = m & (cur < vidx)
                            plsc.store_scatter(
                                wslab, [li_safe], jnp.maximum(cur, vidx),
                                mask=need,
                            )
                            return jnp.any(need)

                        lax.while_loop(lambda c3: c3, body, jnp.bool_(True))

                    return c2

                lax.fori_loop(0, CCH // L, grp, 0)
            return carry

        lax.fori_loop(0, NCCH // 2, coord_pair, 0)

        @plsc.parallel_loop(0, SLAB, 8 * L, unroll=2)
        def _(o0):
            for u in range(4):
                o = o0 + u * 2 * L
                w0 = wslab[pl.ds(o, L)]
                w1 = wslab[pl.ds(o + L, L)]
                w0 = jnp.where(w0 < 0, V, w0)
                w1 = jnp.where(w1 < 0, V, w1)
                wpack[pl.ds(o, 2 * L)] = plsc.pack(
                    w0, w1, format=plsc.PackFormat.INTERLEAVED
                )

        pltpu.sync_copy(wpack, wshared.at[pl.ds(base, SLAB)])

        plsc.subcore_barrier()

        pltpu.make_async_copy(featT_hbm.at[b, ch0], ft0, sem_ft).wait()
        pltpu.make_async_copy(featT_hbm.at[b, ch0 + 1], ft1, sem_ft).wait()

        pltpu.async_copy(wshared.at[pl.ds(0, CHUNK)], wbuf.at[0], sem_w)

        def emit_pair(q, carry):
            for s in range(2):
                k = q * 2 + s
                off = k * CHUNK
                t = b * NCHUNK + k
                pltpu.make_async_copy(
                    wshared.at[pl.ds(0, CHUNK)], wbuf.at[s], sem_w
                ).wait()

                @pl.when(k < NCHUNK - 1)
                def _():
                    pltpu.async_copy(
                        wshared.at[pl.ds(off + CHUNK, CHUNK)],
                        wbuf.at[1 - s],
                        sem_w,
                    )

                @pl.when(t >= 2)
                def _():
                    pltpu.make_async_copy(
                        obuf.at[s, 0], out_hbm.at[b, ch0, pl.ds(off, CHUNK)],
                        sem_o,
                    ).wait()
                    pltpu.make_async_copy(
                        obuf.at[s, 1], out_hbm.at[b, ch0, pl.ds(off, CHUNK)],
                        sem_o,
                    ).wait()

                @plsc.parallel_loop(0, CHUNK, 8 * L, unroll=2)
                def _(o0):
                    for u in range(4):
                        o = o0 + u * 2 * L
                        w16 = wbuf[s, pl.ds(o, 2 * L)]
                        g0, g1 = plsc.unpack(
                            w16, format=plsc.PackFormat.INTERLEAVED
                        )
                        obuf[s, 0, pl.ds(o, L)] = plsc.load_gather(ft0, [g0])
                        obuf[s, 0, pl.ds(o + L, L)] = plsc.load_gather(
                            ft0, [g1]
                        )
                        obuf[s, 1, pl.ds(o, L)] = plsc.load_gather(ft1, [g0])
                        obuf[s, 1, pl.ds(o + L, L)] = plsc.load_gather(
                            ft1, [g1]
                        )

                pltpu.async_copy(
                    obuf.at[s, 0], out_hbm.at[b, ch0, pl.ds(off, CHUNK)],
                    sem_o,
                )
                pltpu.async_copy(
                    obuf.at[s, 1], out_hbm.at[b, ch0 + 1, pl.ds(off, CHUNK)],
                    sem_o,
                )
            return carry

        lax.fori_loop(0, NCHUNK // 2, emit_pair, 0)

        if b == 0:
            pltpu.async_copy(featT_hbm.at[1, ch0], ft0, sem_ft)
            pltpu.async_copy(featT_hbm.at[1, ch0 + 1], ft1, sem_ft)
            plsc.subcore_barrier()

    for _ in range(4):
        pltpu.make_async_copy(
            obuf.at[0, 0], out_hbm.at[B - 1, ch0, pl.ds(0, CHUNK)], sem_o
        ).wait()


def kernel(pillar_features, coords):
    feat_pad = jnp.pad(pillar_features, ((0, 0), (0, VPAD - V), (0, 0)))
    featT = _transpose(feat_pad)
    out = _scatter_kernel(featT, coords.reshape(B, V * 3))
    return out.reshape(B, C, Y, X)


# speedup vs baseline: 1.6012x; 1.0090x over previous
import functools

import jax
import jax.numpy as jnp
from jax import lax
from jax.experimental import pallas as pl
from jax.experimental.pallas import tpu as pltpu
from jax.experimental.pallas import tpu_sc as plsc

X = 512
Y = 512
NCELL = X * Y
B, V, C = 2, 20000, 64
VPAD = 20008
L = 16
NC, NS = 2, 16
SLAB = NCELL // NS
CCH = 2000
NCCH = V // CCH
CHUNK = 8192
NCHUNK = NCELL // CHUNK

_mesh = plsc.VectorSubcoreMesh(
    core_axis_name="c", subcore_axis_name="s", num_cores=NC, num_subcores=NS
)
_sc_params = pltpu.CompilerParams(
    needs_layout_passes=False, use_tc_tiling_on_sc=False
)


def _transpose_body(f_ref, o_ref):
    o_ref[...] = f_ref[...].T


def _transpose(feat_pad):
    return pl.pallas_call(
        _transpose_body,
        grid=(B,),
        in_specs=[pl.BlockSpec((None, VPAD, C), lambda b: (b, 0, 0))],
        out_specs=pl.BlockSpec((None, C, VPAD), lambda b: (b, 0, 0)),
        out_shape=jax.ShapeDtypeStruct((B, C, VPAD), jnp.float32),
    )(feat_pad)


@functools.partial(
    pl.kernel,
    out_type=jax.ShapeDtypeStruct((B, C, NCELL), jnp.float32),
    mesh=_mesh,
    compiler_params=_sc_params,
    scratch_types=[
        pltpu.VMEM((2, CCH * 3), jnp.int32),
        pltpu.VMEM((SLAB,), jnp.int32),
        pltpu.VMEM((SLAB,), jnp.int16),
        pltpu.VMEM((VPAD,), jnp.float32),
        pltpu.VMEM((VPAD,), jnp.float32),
        pltpu.VMEM((2, CHUNK), jnp.int16),
        pltpu.VMEM((2, 2, CHUNK), jnp.float32),
        pltpu.VMEM_SHARED((NCELL,), jnp.int16),
        pltpu.VMEM_SHARED((V * 3,), jnp.int32),
        pltpu.SemaphoreType.DMA,
        pltpu.SemaphoreType.DMA,
        pltpu.SemaphoreType.DMA,
        pltpu.SemaphoreType.DMA,
    ],
)
def _scatter_kernel(
    featT_hbm, coords_hbm, out_hbm,
    cbuf, wslab, wpack, ft0, ft1, wbuf, obuf, wshared, cshared,
    sem_c, sem_ft, sem_w, sem_o,
):
    cid = lax.axis_index("c")
    sid = lax.axis_index("s")
    ch0 = 2 * (cid * NS + sid)
    base = sid * SLAB
    lanes = lax.iota(jnp.int32, L)

    pltpu.async_copy(featT_hbm.at[0, ch0], ft0, sem_ft)
    pltpu.async_copy(featT_hbm.at[0, ch0 + 1], ft1, sem_ft)

    for b in range(B):
        @pl.when(sid < 12)
        def _():
            pltpu.sync_copy(
                coords_hbm.at[b, pl.ds(sid * 5000, 5000)],
                cshared.at[pl.ds(sid * 5000, 5000)],
            )

        @plsc.parallel_loop(0, SLAB, 8 * L, unroll=2)
        def _(o0):
            for u in range(8):
                wslab[pl.ds(o0 + u * L, L)] = jnp.full((L,), -1, jnp.int32)

        plsc.subcore_barrier()
        pltpu.async_copy(cshared.at[pl.ds(0, CCH * 3)], cbuf.at[0], sem_c)

        def coord_pair(q, carry):
            for s in range(2):
                k = q * 2 + s
                pltpu.make_async_copy(
                    cshared.at[pl.ds(0, CCH * 3)], cbuf.at[s], sem_c
                ).wait()

                @pl.when(k < NCCH - 1)
                def _():
                    pltpu.async_copy(
                        cshared.at[pl.ds((k + 1) * (CCH * 3), CCH * 3)],
                        cbuf.at[1 - s],
                        sem_c,
                    )

                def grp(i, c2):
                    vidx3 = (i * L + lanes) * 3
                    xs = plsc.load_gather(cbuf.at[s], [vidx3])
                    ys = plsc.load_gather(cbuf.at[s], [vidx3 + 1])
                    li = ys * X + xs - base
                    m = (li >= 0) & (li < SLAB)
                    li_safe = jnp.clip(li, 0, SLAB - 1)
                    vidx = (k * CCH + i * L) + lanes

                    @pl.when(jnp.any(m))
                    def _():
                        def body(_):
                            cur = plsc.load_gather(wslab, [li_safe])
                            need = m & (cur < vidx)
                            plsc.store_scatter(
                                wslab, [li_safe], jnp.maximum(cur, vidx),
                                mask=need,
                            )
                            return jnp.any(need)

                        lax.while_loop(lambda c3: c3, body, jnp.bool_(True))

                    return c2

                lax.fori_loop(0, CCH // L, grp, 0)
            return carry

        lax.fori_loop(0, NCCH // 2, coord_pair, 0)

        @plsc.parallel_loop(0, SLAB, 8 * L, unroll=2)
        def _(o0):
            for u in range(4):
                o = o0 + u * 2 * L
                w0 = wslab[pl.ds(o, L)]
                w1 = wslab[pl.ds(o + L, L)]
                w0 = jnp.where(w0 < 0, V, w0)
                w1 = jnp.where(w1 < 0, V, w1)
                wpack[pl.ds(o, 2 * L)] = plsc.pack(
                    w0, w1, format=plsc.PackFormat.INTERLEAVED
                )

        pltpu.sync_copy(wpack, wshared.at[pl.ds(base, SLAB)])

        plsc.subcore_barrier()

        pltpu.make_async_copy(featT_hbm.at[b, ch0], ft0, sem_ft).wait()
        pltpu.make_async_copy(featT_hbm.at[b, ch0 + 1], ft1, sem_ft).wait()

        pltpu.async_copy(wshared.at[pl.ds(0, CHUNK)], wbuf.at[0], sem_w)

        def emit_pair(q, carry):
            for s in range(2):
                k = q * 2 + s
                off = k * CHUNK
                t = b * NCHUNK + k
                pltpu.make_async_copy(
                    wshared.at[pl.ds(0, CHUNK)], wbuf.at[s], sem_w
                ).wait()

                @pl.when(k < NCHUNK - 1)
                def _():
                    pltpu.async_copy(
                        wshared.at[pl.ds(off + CHUNK, CHUNK)],
                        wbuf.at[1 - s],
                        sem_w,
                    )

                @pl.when(t >= 2)
                def _():
                    pltpu.make_async_copy(
                        obuf.at[s, 0], out_hbm.at[b, ch0, pl.ds(off, CHUNK)],
                        sem_o,
                    ).wait()
                    pltpu.make_async_copy(
                        obuf.at[s, 1], out_hbm.at[b, ch0, pl.ds(off, CHUNK)],
                        sem_o,
                    ).wait()

                @plsc.parallel_loop(0, CHUNK, 8 * L, unroll=2)
                def _(o0):
                    for u in range(4):
                        o = o0 + u * 2 * L
                        w16 = wbuf[s, pl.ds(o, 2 * L)]
                        g0, g1 = plsc.unpack(
                            w16, format=plsc.PackFormat.INTERLEAVED
                        )
                        obuf[s, 0, pl.ds(o, L)] = plsc.load_gather(ft0, [g0])
                        obuf[s, 0, pl.ds(o + L, L)] = plsc.load_gather(
                            ft0, [g1]
                        )
                        obuf[s, 1, pl.ds(o, L)] = plsc.load_gather(ft1, [g0])
                        obuf[s, 1, pl.ds(o + L, L)] = plsc.load_gather(
                            ft1, [g1]
                        )

                pltpu.async_copy(
                    obuf.at[s, 0], out_hbm.at[b, ch0, pl.ds(off, CHUNK)],
                    sem_o,
                )
                pltpu.async_copy(
                    obuf.at[s, 1], out_hbm.at[b, ch0 + 1, pl.ds(off, CHUNK)],
                    sem_o,
                )
            return carry

        lax.fori_loop(0, NCHUNK // 2, emit_pair, 0)

        if b == 0:
            pltpu.async_copy(featT_hbm.at[1, ch0], ft0, sem_ft)
            pltpu.async_copy(featT_hbm.at[1, ch0 + 1], ft1, sem_ft)
            plsc.subcore_barrier()

    for _ in range(4):
        pltpu.make_async_copy(
            obuf.at[0, 0], out_hbm.at[B - 1, ch0, pl.ds(0, CHUNK)], sem_o
        ).wait()


def kernel(pillar_features, coords):
    feat_pad = jnp.pad(pillar_features, ((0, 0), (0, VPAD - V), (0, 0)))
    featT = _transpose(feat_pad)
    out = _scatter_kernel(featT, coords.reshape(B, V * 3))
    return out.reshape(B, C, Y, X)
